# baseline (speedup 1.0000x reference)
import jax
import jax.numpy as jnp
from jax import lax
from jax.experimental import pallas as pl

G0, G1 = 64, 128
KN = G0 * G1
TN = 256


def _wprep_kernel(w_ref, wh_ref, wl_ref, w2_ref, t_ref):
    wf = w_ref[...]
    wh = wf.astype(jnp.bfloat16)
    wh_ref[...] = wh
    wl_ref[...] = (wf - wh.astype(jnp.float32)).astype(jnp.bfloat16)
    w2_ref[...] = jnp.sum(wf * wf, axis=0, keepdims=True)
    ki = lax.broadcasted_iota(jnp.int32, (KN, 128), 0)
    col = lax.broadcasted_iota(jnp.int32, (KN, 128), 1)
    tv = jnp.where(col == 0, ki // G1, jnp.where(col == 1, ki % G1, 0))
    t_ref[...] = tv.astype(jnp.bfloat16)


def _som_kernel(x_ref, wh_ref, wl_ref, w2_ref, t_ref, out_ref):
    x = x_ref[...]
    x2 = jnp.sum(x * x, axis=1, keepdims=True)
    xs = -2.0 * x
    xh = xs.astype(jnp.bfloat16)
    xl = (xs - xh.astype(jnp.float32)).astype(jnp.bfloat16)
    dn = (((1,), (0,)), ((), ()))
    wh, wl = wh_ref[...], wl_ref[...]
    dot = (lax.dot_general(xh, wh, dn, preferred_element_type=jnp.float32)
           + lax.dot_general(xh, wl, dn, preferred_element_type=jnp.float32)
           + lax.dot_general(xl, wh, dn, preferred_element_type=jnp.float32))
    a = dot + w2_ref[...]
    m = jnp.min(a, axis=1, keepdims=True)
    onehot = (a == m).astype(jnp.bfloat16)
    pos = lax.dot_general(onehot, t_ref[...], dn,
                          preferred_element_type=jnp.float32)
    p0 = jnp.clip(pos[:, 0:1], 0.0, float(G0 - 1))
    p1 = jnp.clip(pos[:, 1:2], 0.0, float(G1 - 1))
    iu = lax.broadcasted_iota(jnp.int32, (TN, G0), 1).astype(jnp.float32)
    iv = lax.broadcasted_iota(jnp.int32, (TN, G1), 1).astype(jnp.float32)
    u = jnp.exp(-((iu - p0) * (iu - p0)))
    v = jnp.exp(-((iv - p1) * (iv - p1)))
    wgt = jnp.concatenate([v * u[:, i:i + 1] for i in range(G0)], axis=1)
    s = jnp.sum(u, axis=1, keepdims=True) * jnp.sum(v, axis=1, keepdims=True)
    out_ref[...] = x2 * s + jnp.sum(wgt * a, axis=1, keepdims=True)


def kernel(x, w):
    n, d = x.shape
    wh, wl, w2, t = pl.pallas_call(
        _wprep_kernel,
        out_shape=(
            jax.ShapeDtypeStruct((d, KN), jnp.bfloat16),
            jax.ShapeDtypeStruct((d, KN), jnp.bfloat16),
            jax.ShapeDtypeStruct((1, KN), jnp.float32),
            jax.ShapeDtypeStruct((KN, 128), jnp.bfloat16),
        ),
    )(w)
    out = pl.pallas_call(
        _som_kernel,
        grid=(n // TN,),
        in_specs=[
            pl.BlockSpec((TN, d), lambda i: (i, 0)),
            pl.BlockSpec((d, KN), lambda i: (0, 0)),
            pl.BlockSpec((d, KN), lambda i: (0, 0)),
            pl.BlockSpec((1, KN), lambda i: (0, 0)),
            pl.BlockSpec((KN, 128), lambda i: (0, 0)),
        ],
        out_specs=pl.BlockSpec((TN, 1), lambda i: (i, 0)),
        out_shape=jax.ShapeDtypeStruct((n, 1), jnp.float32),
    )(x, wh, wl, w2, t)
    return out[:, 0]


# speedup vs baseline: 1.0269x; 1.0269x over previous
import jax
import jax.numpy as jnp
from jax import lax
from jax.experimental import pallas as pl
from jax.experimental.pallas import tpu as pltpu

G0, G1 = 64, 128
KN = G0 * G1
TN = 256


def _som_kernel(x_ref, w_ref, out_ref, wh_ref, wl_ref, w2_ref, t_ref):
    wf = w_ref[...]
    wh = wf.astype(jnp.bfloat16)
    wh_ref[...] = wh
    wl_ref[...] = (wf - wh.astype(jnp.float32)).astype(jnp.bfloat16)
    w2_ref[...] = jnp.sum(wf * wf, axis=0, keepdims=True)
    ki = lax.broadcasted_iota(jnp.int32, (KN, 128), 0)
    col = lax.broadcasted_iota(jnp.int32, (KN, 128), 1)
    tv = jnp.where(col == 0, ki // G1, jnp.where(col == 1, ki % G1, 0))
    t_ref[...] = tv.astype(jnp.bfloat16)

    dn = (((1,), (0,)), ((), ()))

    def body(tile, carry):
        x = x_ref[pl.ds(tile * TN, TN), :]
        x2 = jnp.sum(x * x, axis=1, keepdims=True)
        xs = -2.0 * x
        xh = xs.astype(jnp.bfloat16)
        xl = (xs - xh.astype(jnp.float32)).astype(jnp.bfloat16)
        whv, wlv = wh_ref[...], wl_ref[...]
        dot = (lax.dot_general(xh, whv, dn, preferred_element_type=jnp.float32)
               + lax.dot_general(xh, wlv, dn, preferred_element_type=jnp.float32)
               + lax.dot_general(xl, whv, dn, preferred_element_type=jnp.float32))
        a = dot + w2_ref[...]
        m = jnp.min(a, axis=1, keepdims=True)
        onehot = (a == m).astype(jnp.bfloat16)
        pos = lax.dot_general(onehot, t_ref[...], dn,
                              preferred_element_type=jnp.float32)
        p0 = jnp.clip(pos[:, 0:1], 0.0, float(G0 - 1))
        p1 = jnp.clip(pos[:, 1:2], 0.0, float(G1 - 1))
        iu = lax.broadcasted_iota(jnp.int32, (TN, G0), 1).astype(jnp.float32)
        iv = lax.broadcasted_iota(jnp.int32, (TN, G1), 1).astype(jnp.float32)
        u = jnp.exp(-((iu - p0) * (iu - p0)))
        v = jnp.exp(-((iv - p1) * (iv - p1)))
        wgt = jnp.concatenate([v * u[:, i:i + 1] for i in range(G0)], axis=1)
        s = (jnp.sum(u, axis=1, keepdims=True)
             * jnp.sum(v, axis=1, keepdims=True))
        out_ref[pl.ds(tile * TN, TN), :] = (
            x2 * s + jnp.sum(wgt * a, axis=1, keepdims=True))
        return carry

    lax.fori_loop(0, x_ref.shape[0] // TN, body, 0)


def kernel(x, w):
    n, d = x.shape
    out = pl.pallas_call(
        _som_kernel,
        out_shape=jax.ShapeDtypeStruct((n, 1), jnp.float32),
        scratch_shapes=[
            pltpu.VMEM((d, KN), jnp.bfloat16),
            pltpu.VMEM((d, KN), jnp.bfloat16),
            pltpu.VMEM((1, KN), jnp.float32),
            pltpu.VMEM((KN, 128), jnp.bfloat16),
        ],
    )(x, w)
    return out[:, 0]
